# baseline (speedup 1.0000x reference)
import functools

import jax
import jax.numpy as jnp
from jax import lax
from jax.experimental import pallas as pl
from jax.experimental.pallas import tpu as pltpu
from jax.experimental.pallas import tpu_sc as plsc

NUM_CORES = 2
NUM_SUBCORES = 16
NUM_WORKERS = NUM_CORES * NUM_SUBCORES
CHUNK = 128


@functools.lru_cache(maxsize=None)
def _make_gather(n_tokens: int, dim: int):
    assert n_tokens % (NUM_WORKERS * CHUNK) == 0
    chunks_per_w = n_tokens // (NUM_WORKERS * CHUNK)
    rows_per_w = chunks_per_w * CHUNK
    mesh = plsc.VectorSubcoreMesh(core_axis_name="c", subcore_axis_name="s")

    def body(idx_hbm, weight_hbm, out_hbm, idx_v, rows_v, gsem):
        wid = lax.axis_index("s") * NUM_CORES + lax.axis_index("c")
        base = wid * rows_per_w
        pltpu.sync_copy(idx_hbm.at[wid], idx_v)

        @pl.loop(0, chunks_per_w)
        def _(g):
            pltpu.async_copy(weight_hbm.at[idx_v.at[g]], rows_v, gsem).wait()
            pltpu.sync_copy(rows_v, out_hbm.at[pl.ds(base + g * CHUNK, CHUNK)])

    return pl.kernel(
        body,
        out_type=jax.ShapeDtypeStruct((n_tokens, dim), jnp.float32),
        mesh=mesh,
        scratch_types=[
            pltpu.VMEM((chunks_per_w, CHUNK), jnp.int32),
            pltpu.VMEM((CHUNK, dim), jnp.float32),
            pltpu.SemaphoreType.DMA,
        ],
        compiler_params=pltpu.CompilerParams(use_tc_tiling_on_sc=False),
    )


def kernel(token_ids, weight):
    batch, seq = token_ids.shape
    _, dim = weight.shape
    n_tokens = batch * seq
    idx = token_ids.reshape(NUM_WORKERS, -1, CHUNK).astype(jnp.int32)
    out = _make_gather(n_tokens, dim)(idx, weight)
    return out.reshape(batch, seq, dim)


# speedup vs baseline: 1.1143x; 1.1143x over previous
import functools

import jax
import jax.numpy as jnp
from jax import lax
from jax.experimental import pallas as pl
from jax.experimental.pallas import tpu as pltpu
from jax.experimental.pallas import tpu_sc as plsc

NUM_CORES = 2
NUM_SUBCORES = 16
NUM_WORKERS = NUM_CORES * NUM_SUBCORES
CHUNK = 128
K = 4
GROUP = K * CHUNK


@functools.lru_cache(maxsize=None)
def _make_gather(n_tokens: int, dim: int):
    assert n_tokens % (NUM_WORKERS * GROUP) == 0
    n_groups = n_tokens // (NUM_WORKERS * GROUP)
    assert n_groups % 2 == 0
    chunks_per_w = n_groups * K
    rows_per_w = chunks_per_w * CHUNK
    mesh = plsc.VectorSubcoreMesh(core_axis_name="c", subcore_axis_name="s")

    def body(idx_hbm, weight_hbm, out_hbm, idx_v, rows0, rows1, gsem,
             wsem0, wsem1):
        wid = lax.axis_index("s") * NUM_CORES + lax.axis_index("c")
        base = wid * rows_per_w
        pltpu.sync_copy(idx_hbm.at[wid], idx_v)
        rows = (rows0, rows1)
        wsem = (wsem0, wsem1)

        def wait_write(b):
            pltpu.make_async_copy(
                rows[b], out_hbm.at[pl.ds(base, GROUP)], wsem[b]).wait()

        @pl.loop(0, n_groups, step=2)
        def _(g0):
            for b in range(2):
                g = g0 + b

                @pl.when(g0 > 0)
                def _():
                    wait_write(b)

                descs = []
                for k in range(K):
                    c = g * K + k
                    descs.append(pltpu.async_copy(
                        weight_hbm.at[idx_v.at[c]],
                        rows[b].at[pl.ds(k * CHUNK, CHUNK)], gsem))
                for d in descs:
                    d.wait()
                pltpu.async_copy(
                    rows[b], out_hbm.at[pl.ds(base + g * GROUP, GROUP)],
                    wsem[b])

        wait_write(0)
        wait_write(1)

    return pl.kernel(
        body,
        out_type=jax.ShapeDtypeStruct((n_tokens, dim), jnp.float32),
        mesh=mesh,
        scratch_types=[
            pltpu.VMEM((chunks_per_w, CHUNK), jnp.int32),
            pltpu.VMEM((GROUP, dim), jnp.float32),
            pltpu.VMEM((GROUP, dim), jnp.float32),
            pltpu.SemaphoreType.DMA,
            pltpu.SemaphoreType.DMA,
            pltpu.SemaphoreType.DMA,
        ],
        compiler_params=pltpu.CompilerParams(use_tc_tiling_on_sc=False),
    )


def kernel(token_ids, weight):
    batch, seq = token_ids.shape
    _, dim = weight.shape
    n_tokens = batch * seq
    idx = token_ids.reshape(NUM_WORKERS, -1, CHUNK).astype(jnp.int32)
    out = _make_gather(n_tokens, dim)(idx, weight)
    return out.reshape(batch, seq, dim)


# speedup vs baseline: 1.1148x; 1.0005x over previous
import functools

import jax
import jax.numpy as jnp
from jax import lax
from jax.experimental import pallas as pl
from jax.experimental.pallas import tpu as pltpu
from jax.experimental.pallas import tpu_sc as plsc

NUM_CORES = 2
NUM_SUBCORES = 16
NUM_WORKERS = NUM_CORES * NUM_SUBCORES
CHUNK = 128
NBUF = 8
LOOKAHEAD = 4


@functools.lru_cache(maxsize=None)
def _make_gather(n_tokens: int, dim: int):
    assert n_tokens % (NUM_WORKERS * CHUNK * NBUF) == 0
    chunks_per_w = n_tokens // (NUM_WORKERS * CHUNK)
    rows_per_w = chunks_per_w * CHUNK
    mesh = plsc.VectorSubcoreMesh(core_axis_name="c", subcore_axis_name="s")

    def body(idx_hbm, weight_hbm, out_hbm, idx_v, rows_v, *sems):
        gsem = sems[:NBUF]
        wsem = sems[NBUF:]
        wid = lax.axis_index("s") * NUM_CORES + lax.axis_index("c")
        base = wid * rows_per_w
        pltpu.sync_copy(idx_hbm.at[wid], idx_v)

        def buf(b):
            return rows_v.at[pl.ds(b * CHUNK, CHUNK)]

        def issue_gather(c, b):
            pltpu.async_copy(weight_hbm.at[idx_v.at[c]], buf(b), gsem[b])

        def wait_gather(b):
            pltpu.make_async_copy(
                weight_hbm.at[idx_v.at[0]], buf(b), gsem[b]).wait()

        def issue_write(c, b):
            pltpu.async_copy(
                buf(b), out_hbm.at[pl.ds(base + c * CHUNK, CHUNK)], wsem[b])

        def wait_write(b):
            pltpu.make_async_copy(
                buf(b), out_hbm.at[pl.ds(base, CHUNK)], wsem[b]).wait()

        for c in range(LOOKAHEAD):
            issue_gather(c, c % NBUF)

        @pl.loop(0, chunks_per_w, step=NBUF)
        def _(c0):
            for b in range(NBUF):
                c = c0 + b
                ba = (b + LOOKAHEAD) % NBUF

                @pl.when(c + LOOKAHEAD < chunks_per_w)
                def _():
                    @pl.when(c + LOOKAHEAD >= NBUF)
                    def _():
                        wait_write(ba)
                    issue_gather(c + LOOKAHEAD, ba)

                wait_gather(b)
                issue_write(c, b)

        for b in range(NBUF):
            wait_write(b)

    return pl.kernel(
        body,
        out_type=jax.ShapeDtypeStruct((n_tokens, dim), jnp.float32),
        mesh=mesh,
        scratch_types=[
            pltpu.VMEM((chunks_per_w, CHUNK), jnp.int32),
            pltpu.VMEM((NBUF * CHUNK, dim), jnp.float32),
        ] + [pltpu.SemaphoreType.DMA] * (2 * NBUF),
        compiler_params=pltpu.CompilerParams(use_tc_tiling_on_sc=False),
    )


def kernel(token_ids, weight):
    batch, seq = token_ids.shape
    _, dim = weight.shape
    n_tokens = batch * seq
    idx = token_ids.reshape(NUM_WORKERS, -1, CHUNK).astype(jnp.int32)
    out = _make_gather(n_tokens, dim)(idx, weight)
    return out.reshape(batch, seq, dim)
